# baseline (speedup 1.0000x reference)
import numpy as np

import jax
import jax.numpy as jnp
from jax.experimental import pallas as pl
from jax.experimental.pallas import tpu as pltpu

P = 8


def _selector(p: int) -> np.ndarray:
    d = 9 * p + 1
    sel = np.zeros((d, 10 * p), np.float32)
    for e in range(p):
        for k in range(4):
            sel[e * 4 + k, e * 10 + k] = 1.0
            sel[4 * p + e * 4 + k, e * 10 + 4 + k] = 1.0
        sel[8 * p + e, e * 10 + 8] = 1.0
        sel[9 * p, e * 10 + 9] = 1.0
    return sel


def _make_stats_body(nblk: int, n_edges: float, p: int):
    def body(sp_ref, dp_ref, ap_ref, w_ref, b_ref, gm_ref, bt_ref, sel_ref,
             kp_ref, o_ref, q_ref):
        i = pl.program_id(0)
        sp = sp_ref[...]
        dp = dp_ref[...]
        ap = ap_ref[...]
        ones = jnp.ones((sp.shape[0], 1), jnp.float32)
        z = jnp.concatenate([sp, dp, ap, ones], axis=1)
        q = jax.lax.dot_general(
            z, z, (((0,), (0,)), ((), ())), preferred_element_type=jnp.float32)

        @pl.when(i == 0)
        def _init():
            q_ref[...] = q

        @pl.when(i != 0)
        def _acc():
            q_ref[...] += q

        @pl.when(i == nblk - 1)
        def _finish():
            qf = q_ref[...]
            g = jnp.zeros((10, 10), jnp.float32)
            for e in range(p):
                ce = sel_ref[:, e * 10:(e + 1) * 10]
                a = jnp.dot(qf, ce, preferred_element_type=jnp.float32)
                g = g + jax.lax.dot_general(
                    ce, a, (((0,), (0,)), ((), ())),
                    preferred_element_type=jnp.float32)
            Wm = w_ref[...]
            bb = b_ref[...]
            G9 = g[0:9, 0:9]
            csum = g[9:10, 0:9]
            cW = jnp.dot(csum, Wm, preferred_element_type=jnp.float32)
            GW = jnp.dot(G9, Wm, preferred_element_type=jnp.float32)
            sumsq = (jnp.sum(Wm * GW, axis=0, keepdims=True)
                     + 2.0 * bb * cW + n_edges * bb * bb)
            mean = (cW + n_edges * bb) / n_edges
            var = sumsq / n_edges - mean * mean
            scale = gm_ref[...] * jax.lax.rsqrt(var + 1e-5)
            bf = (bb - mean) * scale + bt_ref[...]
            kp = kp_ref[...]
            o_ref[0:1, :] = jnp.dot(scale, kp,
                                    preferred_element_type=jnp.float32)
            o_ref[1:2, :] = jnp.dot(bf, kp,
                                    preferred_element_type=jnp.float32)

    return body


def _main_body(sb_ref, ws_ref, wd_ref, wa_ref, sp_ref, dp_ref, ap_ref, o_ref):
    acc = jnp.dot(sp_ref[...], ws_ref[...],
                  preferred_element_type=jnp.float32)
    acc += jnp.dot(dp_ref[...], wd_ref[...],
                   preferred_element_type=jnp.float32)
    acc += jnp.dot(ap_ref[...], wa_ref[...],
                   preferred_element_type=jnp.float32)
    o_ref[...] = acc * sb_ref[0:1, :] + sb_ref[1:2, :]


def kernel(src, dest, edge_attr, W, b, gamma, beta):
    E = src.shape[0]
    R = E // P
    D = 9 * P + 1
    BRS = 4000
    BRM = 2000
    nblk_s = R // BRS
    nblk_m = R // BRM

    sp = src.reshape(R, 4 * P)
    dp = dest.reshape(R, 4 * P)
    ap = edge_attr.reshape(R, P)

    eye = jnp.eye(P, dtype=jnp.float32)
    ws = jnp.kron(eye, W[0:4, :])
    wd = jnp.kron(eye, W[4:8, :])
    wa = jnp.kron(eye, W[8:9, :])
    kp = jnp.kron(jnp.ones((1, P), jnp.float32), jnp.eye(84, dtype=jnp.float32))
    sel = jnp.asarray(_selector(P))
    b2 = b.reshape(1, 84)
    gm2 = gamma.reshape(1, 84)
    bt2 = beta.reshape(1, 84)

    const = lambda i: (0, 0)
    row = lambda i: (i, 0)

    sb = pl.pallas_call(
        _make_stats_body(nblk_s, float(E), P),
        grid=(nblk_s,),
        in_specs=[
            pl.BlockSpec((BRS, 4 * P), row),
            pl.BlockSpec((BRS, 4 * P), row),
            pl.BlockSpec((BRS, P), row),
            pl.BlockSpec((9, 84), const),
            pl.BlockSpec((1, 84), const),
            pl.BlockSpec((1, 84), const),
            pl.BlockSpec((1, 84), const),
            pl.BlockSpec((D, 10 * P), const),
            pl.BlockSpec((84, 84 * P), const),
        ],
        out_specs=pl.BlockSpec((2, 84 * P), const),
        out_shape=jax.ShapeDtypeStruct((2, 84 * P), jnp.float32),
        scratch_shapes=[pltpu.VMEM((D, D), jnp.float32)],
    )(sp, dp, ap, W, b2, gm2, bt2, sel, kp)

    out2 = pl.pallas_call(
        _main_body,
        grid=(nblk_m,),
        in_specs=[
            pl.BlockSpec((2, 84 * P), const),
            pl.BlockSpec((4 * P, 84 * P), const),
            pl.BlockSpec((4 * P, 84 * P), const),
            pl.BlockSpec((P, 84 * P), const),
            pl.BlockSpec((BRM, 4 * P), row),
            pl.BlockSpec((BRM, 4 * P), row),
            pl.BlockSpec((BRM, P), row),
        ],
        out_specs=pl.BlockSpec((BRM, 84 * P), row),
        out_shape=jax.ShapeDtypeStruct((R, 84 * P), jnp.float32),
    )(sb, ws, wd, wa, sp, dp, ap)
    return out2.reshape(E, 84)


# speedup vs baseline: 16.8263x; 16.8263x over previous
import jax
import jax.numpy as jnp
from jax.experimental import pallas as pl
from jax.experimental.pallas import tpu as pltpu


def _make_stats_body(nblk: int, n_edges: float):
    def body(x_ref, wt_ref, b_ref, gm_ref, bt_ref, o_ref, g_ref):
        i = pl.program_id(0)
        blk = x_ref[...]
        g = jax.lax.dot_general(
            blk, blk, (((1,), (1,)), ((), ())),
            preferred_element_type=jnp.float32)

        @pl.when(i == 0)
        def _init():
            g_ref[...] = g

        @pl.when(i != 0)
        def _acc():
            g_ref[...] += g

        @pl.when(i == nblk - 1)
        def _finish():
            gf = g_ref[...]
            G9 = gf[0:9, 0:9]
            csumT = gf[0:9, 9:10]
            WT = wt_ref[...]
            bT = b_ref[...]
            WG = jnp.dot(WT, G9, preferred_element_type=jnp.float32)
            sumsqT = (jnp.sum(WG * WT, axis=1, keepdims=True))
            cWT = jnp.dot(WT, csumT, preferred_element_type=jnp.float32)
            sumsqT = sumsqT + 2.0 * bT * cWT + n_edges * bT * bT
            meanT = (cWT + n_edges * bT) / n_edges
            varT = sumsqT / n_edges - meanT * meanT
            scaleT = gm_ref[...] * jax.lax.rsqrt(varT + 1e-5)
            bfT = (bT - meanT) * scaleT + bt_ref[...]
            o_ref[:, 0:1] = scaleT
            o_ref[:, 1:2] = bfT

    return body


def _main_body(sb_ref, wall_ref, x_ref, o_ref):
    acc = jnp.dot(wall_ref[...], x_ref[...],
                  preferred_element_type=jnp.float32)
    o_ref[...] = acc * sb_ref[:, 0:1] + sb_ref[:, 1:2]


def kernel(src, dest, edge_attr, W, b, gamma, beta):
    E = src.shape[0]
    BLS = 64000
    BLM = 32000
    nblk_s = E // BLS
    nblk_m = E // BLM

    xT = jnp.concatenate(
        [src.T, dest.T, edge_attr.T, jnp.ones((1, E), jnp.float32)], axis=0)
    wall = jnp.concatenate([W.T, jnp.zeros((84, 1), jnp.float32)], axis=1)

    const = lambda i: (0, 0)
    col = lambda i: (0, i)

    sb = pl.pallas_call(
        _make_stats_body(nblk_s, float(E)),
        grid=(nblk_s,),
        in_specs=[
            pl.BlockSpec((10, BLS), col),
            pl.BlockSpec((84, 9), const),
            pl.BlockSpec((84, 1), const),
            pl.BlockSpec((84, 1), const),
            pl.BlockSpec((84, 1), const),
        ],
        out_specs=pl.BlockSpec((84, 2), const),
        out_shape=jax.ShapeDtypeStruct((84, 2), jnp.float32),
        scratch_shapes=[pltpu.VMEM((10, 10), jnp.float32)],
    )(xT, W.T, b.reshape(84, 1), gamma.reshape(84, 1), beta.reshape(84, 1))

    outT = pl.pallas_call(
        _main_body,
        grid=(nblk_m,),
        in_specs=[
            pl.BlockSpec((84, 2), const),
            pl.BlockSpec((84, 10), const),
            pl.BlockSpec((10, BLM), col),
        ],
        out_specs=pl.BlockSpec((84, BLM), col),
        out_shape=jax.ShapeDtypeStruct((84, E), jnp.float32),
    )(sb, wall, xT)
    return outT.T
